# Initial kernel scaffold; baseline (speedup 1.0000x reference)
import jax
import jax.numpy as jnp
from jax.experimental import pallas as pl


def kernel(sent1, sent2, lsize_list, rsize_list, emb_table, W, b):
    raise NotImplementedError("write your pallas kernel here")


# speedup vs baseline: 10.9982x; 10.9982x over previous
import functools

import jax
import jax.numpy as jnp
from jax import lax
from jax.experimental import pallas as pl
from jax.experimental.pallas import tpu as pltpu
from jax.experimental.pallas import tpu_sc as plsc

B, L, V, D = 4096, 50, 100000, 128
NC, NS = 2, 16
NW = NC * NS
RPW = B // NW
IPW = RPW * L
LANES = 16
GRP = RPW // LANES

VB = 2000


def _proj_body(emb_ref, wt_ref, q1_ref, q2_ref):
    r = jnp.dot(emb_ref[...], wt_ref[...], preferred_element_type=jnp.float32)
    q1_ref[...] = r[:, 0:1]
    q2_ref[...] = r[:, 1:2]


_proj_call = pl.pallas_call(
    _proj_body,
    grid=(V // VB,),
    in_specs=[
        pl.BlockSpec((VB, D), lambda i: (i, 0)),
        pl.BlockSpec((D, 8), lambda i: (0, 0)),
    ],
    out_specs=[
        pl.BlockSpec((VB, 1), lambda i: (i, 0)),
        pl.BlockSpec((VB, 1), lambda i: (i, 0)),
    ],
    out_shape=[
        jax.ShapeDtypeStruct((V, 1), jnp.float32),
        jax.ShapeDtypeStruct((V, 1), jnp.float32),
    ],
)


@functools.partial(
    pl.kernel,
    out_type=jax.ShapeDtypeStruct((2 * B,), jnp.float32),
    mesh=plsc.VectorSubcoreMesh(core_axis_name="c", subcore_axis_name="s"),
    scratch_types=[
        pltpu.VMEM((IPW,), jnp.int32),
        pltpu.VMEM((IPW,), jnp.int32),
        pltpu.VMEM((IPW,), jnp.int32),
        pltpu.VMEM((IPW,), jnp.float32),
        pltpu.VMEM((IPW,), jnp.float32),
        pltpu.VMEM((RPW,), jnp.float32),
        pltpu.VMEM((RPW,), jnp.float32),
        pltpu.VMEM_SHARED((NS * RPW,), jnp.float32),
        pltpu.VMEM_SHARED((NS * RPW,), jnp.float32),
        pltpu.VMEM((RPW,), jnp.int32),
        pltpu.VMEM((RPW,), jnp.int32),
        pltpu.VMEM((LANES,), jnp.float32),
        pltpu.VMEM((RPW,), jnp.float32),
        pltpu.VMEM((RPW,), jnp.float32),
        pltpu.SemaphoreType.DMA,
        pltpu.SemaphoreType.DMA,
    ],
)
def _sc_pool(q1_hbm, q2_hbm, idx1_hbm, idx2_hbm, seg_hbm, ls_hbm, rs_hbm,
             bias_hbm, zero_hbm, out_hbm,
             idx1_v, idx2_v, seg_v, rows1_v, rows2_v, acc1_v, acc2_v,
             acc1_sh, acc2_sh, ls_v, rs_v, bias_v, out0_v, out1_v, sem1, sem2):
    sid = lax.axis_index("s")
    wid = sid * NC + lax.axis_index("c")
    base = wid * RPW
    ibase = wid * IPW
    sbase = sid * RPW

    pltpu.sync_copy(idx1_hbm.at[pl.ds(ibase, IPW)], idx1_v)
    pltpu.sync_copy(idx2_hbm.at[pl.ds(ibase, IPW)], idx2_v)
    cp1 = pltpu.async_copy(q1_hbm.at[idx1_v], rows1_v, sem1)
    cp2 = pltpu.async_copy(q2_hbm.at[idx2_v], rows2_v, sem2)
    pltpu.sync_copy(seg_hbm.at[sid], seg_v)
    pltpu.sync_copy(ls_hbm.at[pl.ds(base, RPW)], ls_v)
    pltpu.sync_copy(rs_hbm.at[pl.ds(base, RPW)], rs_v)
    pltpu.sync_copy(bias_hbm, bias_v)
    pltpu.sync_copy(zero_hbm, acc1_sh.at[pl.ds(sbase, RPW)])
    pltpu.sync_copy(zero_hbm, acc2_sh.at[pl.ds(sbase, RPW)])
    cp1.wait()
    pltpu.sync_copy(rows1_v, acc1_sh.at[seg_v], add=True)
    cp2.wait()
    pltpu.sync_copy(rows2_v, acc2_sh.at[seg_v], add=True)
    pltpu.sync_copy(acc1_sh.at[pl.ds(sbase, RPW)], acc1_v)
    pltpu.sync_copy(acc2_sh.at[pl.ds(sbase, RPW)], acc2_v)

    bd = bias_v[...]
    for g in range(GRP):
        s1 = acc1_v[pl.ds(g * LANES, LANES)]
        s2 = acc2_v[pl.ds(g * LANES, LANES)]
        rl = 1.0 / ls_v[pl.ds(g * LANES, LANES)].astype(jnp.float32)
        rr = 1.0 / rs_v[pl.ds(g * LANES, LANES)].astype(jnp.float32)
        dd = s1 * rl + s2 * rr + bd
        r1 = 1.0 / (1.0 + jnp.exp(-dd))
        out0_v[pl.ds(g * LANES, LANES)] = 1.0 - r1
        out1_v[pl.ds(g * LANES, LANES)] = r1
    pltpu.sync_copy(out0_v, out_hbm.at[pl.ds(base, RPW)])
    pltpu.sync_copy(out1_v, out_hbm.at[pl.ds(B + base, RPW)])


def kernel(sent1, sent2, lsize_list, rsize_list, emb_table, W, b):
    wdiff = W[1] - W[0]
    wt = jnp.stack([wdiff[:D], wdiff[D:]], axis=1)
    wt = jnp.pad(wt, ((0, 0), (0, 6)))
    q1, q2 = _proj_call(emb_table, wt)

    seg = (jnp.repeat(jnp.arange(RPW, dtype=jnp.int32), L)[None, :]
           + jnp.arange(NS, dtype=jnp.int32)[:, None] * RPW)
    bias_d = jnp.full((LANES,), b[1] - b[0], jnp.float32)
    zacc = jnp.zeros((RPW,), jnp.float32)
    out = _sc_pool(q1.reshape(V), q2.reshape(V),
                   sent1.reshape(-1), sent2.reshape(-1), seg,
                   lsize_list, rsize_list, bias_d, zacc)
    return out.reshape(2, B).T


# speedup vs baseline: 24.4240x; 2.2207x over previous
import functools

import jax
import jax.numpy as jnp
from jax import lax
from jax.experimental import pallas as pl
from jax.experimental.pallas import tpu as pltpu
from jax.experimental.pallas import tpu_sc as plsc

B, L, V, D = 4096, 50, 100000, 128
NC, NS = 2, 16
NW = NC * NS
RPW = B // NW
IPW = RPW * L
LANES = 16
GRP = RPW // LANES

VB = 4096


def _proj_body(emb_ref, wt_ref, q1_ref, q2_ref):
    r = lax.dot_general(wt_ref[...], emb_ref[...],
                        dimension_numbers=(((1,), (1,)), ((), ())),
                        preferred_element_type=jnp.float32)
    q1_ref[...] = r[0]
    q2_ref[...] = r[1]


_proj_call = pl.pallas_call(
    _proj_body,
    grid=((V + VB - 1) // VB,),
    in_specs=[
        pl.BlockSpec((VB, D), lambda i: (i, 0)),
        pl.BlockSpec((8, D), lambda i: (0, 0)),
    ],
    out_specs=[
        pl.BlockSpec((VB,), lambda i: (i,)),
        pl.BlockSpec((VB,), lambda i: (i,)),
    ],
    out_shape=[
        jax.ShapeDtypeStruct((V,), jnp.float32),
        jax.ShapeDtypeStruct((V,), jnp.float32),
    ],
)


@functools.partial(
    pl.kernel,
    out_type=jax.ShapeDtypeStruct((2 * B,), jnp.float32),
    mesh=plsc.VectorSubcoreMesh(core_axis_name="c", subcore_axis_name="s"),
    scratch_types=[
        pltpu.VMEM((IPW,), jnp.int32),
        pltpu.VMEM((IPW,), jnp.int32),
        pltpu.VMEM((IPW,), jnp.float32),
        pltpu.VMEM((IPW,), jnp.float32),
        pltpu.VMEM((RPW,), jnp.int32),
        pltpu.VMEM((RPW,), jnp.int32),
        pltpu.VMEM((LANES,), jnp.float32),
        pltpu.VMEM((RPW,), jnp.float32),
        pltpu.VMEM((RPW,), jnp.float32),
        pltpu.SemaphoreType.DMA,
        pltpu.SemaphoreType.DMA,
    ],
)
def _sc_pool(q1_hbm, q2_hbm, idx1_hbm, idx2_hbm, ls_hbm, rs_hbm,
             bias_hbm, out_hbm,
             idx1_v, idx2_v, rows1_v, rows2_v,
             ls_v, rs_v, bias_v, out0_v, out1_v, sem1, sem2):
    wid = lax.axis_index("s") * NC + lax.axis_index("c")
    base = wid * RPW
    ibase = wid * IPW

    pltpu.sync_copy(idx1_hbm.at[pl.ds(ibase, IPW)], idx1_v)
    cp1 = pltpu.async_copy(q1_hbm.at[idx1_v], rows1_v, sem1)
    pltpu.sync_copy(idx2_hbm.at[pl.ds(ibase, IPW)], idx2_v)
    cp2 = pltpu.async_copy(q2_hbm.at[idx2_v], rows2_v, sem2)
    pltpu.sync_copy(ls_hbm.at[pl.ds(base, RPW)], ls_v)
    pltpu.sync_copy(rs_hbm.at[pl.ds(base, RPW)], rs_v)
    pltpu.sync_copy(bias_hbm, bias_v)

    bd = bias_v[...]
    cp1.wait()
    s1 = [None] * GRP
    for g in range(GRP):
        acc = rows1_v[pl.ds(g * LANES, LANES)]
        for t in range(1, L):
            acc = acc + rows1_v[pl.ds(t * RPW + g * LANES, LANES)]
        s1[g] = acc
    cp2.wait()
    for g in range(GRP):
        acc = rows2_v[pl.ds(g * LANES, LANES)]
        for t in range(1, L):
            acc = acc + rows2_v[pl.ds(t * RPW + g * LANES, LANES)]
        rl = 1.0 / ls_v[pl.ds(g * LANES, LANES)].astype(jnp.float32)
        rr = 1.0 / rs_v[pl.ds(g * LANES, LANES)].astype(jnp.float32)
        dd = s1[g] * rl + acc * rr + bd
        r1 = 1.0 / (1.0 + jnp.exp(-dd))
        out0_v[pl.ds(g * LANES, LANES)] = 1.0 - r1
        out1_v[pl.ds(g * LANES, LANES)] = r1
    pltpu.sync_copy(out0_v, out_hbm.at[pl.ds(base, RPW)])
    pltpu.sync_copy(out1_v, out_hbm.at[pl.ds(B + base, RPW)])


def _wtr(sent):
    return sent.reshape(NW, RPW, L).transpose(0, 2, 1).reshape(-1)


def kernel(sent1, sent2, lsize_list, rsize_list, emb_table, W, b):
    wdiff = W[1] - W[0]
    wt = jnp.stack([wdiff[:D], wdiff[D:]], axis=0)
    wt = jnp.pad(wt, ((0, 6), (0, 0)))
    q1, q2 = _proj_call(emb_table, wt)

    bias_d = jnp.full((LANES,), b[1] - b[0], jnp.float32)
    out = _sc_pool(q1, q2, _wtr(sent1), _wtr(sent2),
                   lsize_list, rsize_list, bias_d)
    return out.reshape(2, B).T


# speedup vs baseline: 27.2978x; 1.1177x over previous
import functools

import jax
import jax.numpy as jnp
from jax import lax
from jax.experimental import pallas as pl
from jax.experimental.pallas import tpu as pltpu
from jax.experimental.pallas import tpu_sc as plsc

B, L, V, D = 4096, 50, 100000, 128
NC, NS = 2, 16
NW = NC * NS
RPW = B // NW
IPW = RPW * L
LANES = 16
GRP = RPW // LANES

VB = 8192


def _proj_body(emb_ref, w_ref, q1_ref, q2_ref):
    w = w_ref[...]
    wd = w[1:2, :] - w[0:1, :]
    lhs = jnp.concatenate([wd[:, :D], wd[:, D:]], axis=0)
    r = lax.dot_general(lhs, emb_ref[...],
                        dimension_numbers=(((1,), (1,)), ((), ())),
                        preferred_element_type=jnp.float32)
    q1_ref[...] = r[0]
    q2_ref[...] = r[1]


_proj_call = pl.pallas_call(
    _proj_body,
    grid=((V + VB - 1) // VB,),
    in_specs=[
        pl.BlockSpec((VB, D), lambda i: (i, 0)),
        pl.BlockSpec((2, 2 * D), lambda i: (0, 0)),
    ],
    out_specs=[
        pl.BlockSpec((VB,), lambda i: (i,)),
        pl.BlockSpec((VB,), lambda i: (i,)),
    ],
    out_shape=[
        jax.ShapeDtypeStruct((V,), jnp.float32),
        jax.ShapeDtypeStruct((V,), jnp.float32),
    ],
)


@functools.partial(
    pl.kernel,
    out_type=jax.ShapeDtypeStruct((2 * B,), jnp.float32),
    mesh=plsc.VectorSubcoreMesh(core_axis_name="c", subcore_axis_name="s"),
    scratch_types=[
        pltpu.VMEM((IPW,), jnp.int32),
        pltpu.VMEM((IPW,), jnp.int32),
        pltpu.VMEM((IPW,), jnp.float32),
        pltpu.VMEM((IPW,), jnp.float32),
        pltpu.VMEM((RPW,), jnp.int32),
        pltpu.VMEM((RPW,), jnp.int32),
        pltpu.VMEM((LANES,), jnp.float32),
        pltpu.VMEM((RPW,), jnp.float32),
        pltpu.VMEM((RPW,), jnp.float32),
        pltpu.SemaphoreType.DMA,
        pltpu.SemaphoreType.DMA,
    ],
)
def _sc_pool(q1_hbm, q2_hbm, idx1_hbm, idx2_hbm, ls_hbm, rs_hbm,
             bias_hbm, out_hbm,
             idx1_v, idx2_v, rows1_v, rows2_v,
             ls_v, rs_v, bias_v, out0_v, out1_v, sem1, sem2):
    wid = lax.axis_index("s") * NC + lax.axis_index("c")
    base = wid * RPW
    ibase = wid * IPW

    pltpu.sync_copy(idx1_hbm.at[pl.ds(ibase, IPW)], idx1_v)
    cp1 = pltpu.async_copy(q1_hbm.at[idx1_v], rows1_v, sem1)
    pltpu.sync_copy(idx2_hbm.at[pl.ds(ibase, IPW)], idx2_v)
    cp2 = pltpu.async_copy(q2_hbm.at[idx2_v], rows2_v, sem2)
    pltpu.sync_copy(ls_hbm.at[pl.ds(base, RPW)], ls_v)
    pltpu.sync_copy(rs_hbm.at[pl.ds(base, RPW)], rs_v)
    pltpu.sync_copy(bias_hbm, bias_v)

    bd = bias_v[...]
    cp1.wait()
    s1 = [None] * GRP
    for g in range(GRP):
        acc = rows1_v[pl.ds(g * LANES, LANES)]
        for t in range(1, L):
            acc = acc + rows1_v[pl.ds(t * RPW + g * LANES, LANES)]
        s1[g] = acc
    cp2.wait()
    for g in range(GRP):
        acc = rows2_v[pl.ds(g * LANES, LANES)]
        for t in range(1, L):
            acc = acc + rows2_v[pl.ds(t * RPW + g * LANES, LANES)]
        rl = 1.0 / ls_v[pl.ds(g * LANES, LANES)].astype(jnp.float32)
        rr = 1.0 / rs_v[pl.ds(g * LANES, LANES)].astype(jnp.float32)
        dd = s1[g] * rl + acc * rr + bd
        r1 = 1.0 / (1.0 + jnp.exp(-dd))
        out0_v[pl.ds(g * LANES, LANES)] = 1.0 - r1
        out1_v[pl.ds(g * LANES, LANES)] = r1
    pltpu.sync_copy(out0_v, out_hbm.at[pl.ds(base, RPW)])
    pltpu.sync_copy(out1_v, out_hbm.at[pl.ds(B + base, RPW)])


def _wtr(sent):
    return sent.reshape(NW, RPW, L).transpose(0, 2, 1).reshape(-1)


def kernel(sent1, sent2, lsize_list, rsize_list, emb_table, W, b):
    q1, q2 = _proj_call(emb_table, W)

    bias_d = jnp.full((LANES,), b[1] - b[0], jnp.float32)
    out = _sc_pool(q1, q2, _wtr(sent1), _wtr(sent2),
                   lsize_list, rsize_list, bias_d)
    return out.reshape(2, B).T
